# baseline (speedup 1.0000x reference)
import functools
import jax
import jax.numpy as jnp
from jax import lax
from jax.experimental import pallas as pl
from jax.experimental.pallas import tpu as pltpu
from jax.experimental.pallas import tpu_sc as plsc

DIM = 64
ENCDIM = 64
OUTD = DIM + ENCDIM
BATCH = 16384

_NC = 2
_NS = 16
_NW = _NC * _NS
_BPW = BATCH // _NW
_L = 16

_mesh = plsc.VectorSubcoreMesh(core_axis_name="c", subcore_axis_name="s")


@functools.partial(
    pl.kernel,
    mesh=_mesh,
    out_type=jax.ShapeDtypeStruct((BATCH, OUTD), jnp.float32),
    scratch_types=[
        pltpu.VMEM((_BPW,), jnp.int32),
        pltpu.VMEM((_BPW,), jnp.int32),
        pltpu.VMEM((_BPW,), jnp.int32),
        pltpu.VMEM((_BPW // 2, 2 * ENCDIM), jnp.float32),
        pltpu.VMEM((_BPW, OUTD), jnp.float32),
        pltpu.SemaphoreType.DMA,
        pltpu.SemaphoreType.DMA,
    ],
)
def _dil_kernel(ids_hbm, prev2_hbm, emb2_hbm, map_hbm, out_hbm,
                ids_v, mid_v, pid_v, prev_v, out_v, sem, sem2):
    wid = lax.axis_index("s") * _NC + lax.axis_index("c")
    base = pl.multiple_of(wid * _BPW, _BPW)
    hbase = pl.multiple_of(wid * (_BPW // 2), _BPW // 2)
    prev_cp = pltpu.async_copy(prev2_hbm.at[pl.ds(hbase, _BPW // 2)],
                               prev_v, sem2)
    pltpu.sync_copy(ids_hbm.at[pl.ds(base, _BPW)], ids_v)
    pltpu.async_copy(map_hbm.at[ids_v], mid_v, sem).wait()

    def _pids(j, carry):
        pid_v[pl.ds(_L * j, _L)] = mid_v[pl.ds(_L * j, _L)] >> 1
        return carry

    lax.fori_loop(0, _BPW // _L, _pids, 0, unroll=4)
    pltpu.async_copy(emb2_hbm.at[pid_v], out_v, sem).wait()
    prev_cp.wait()

    def _grp(j, carry):
        m16 = mid_v[pl.ds(_L * j, _L)]
        for r2 in range(_L):
            r = _L * j + r2
            off = (m16[r2] & 1) * DIM
            for k in range(DIM // _L):
                out_v[r, pl.ds(_L * k, _L)] = out_v[r, pl.ds(off + _L * k, _L)]
            poff = (r2 & 1) * ENCDIM
            prow = (_L // 2) * j + (r2 >> 1)
            for k in range(ENCDIM // _L):
                out_v[r, pl.ds(DIM + _L * k, _L)] = prev_v[
                    prow, pl.ds(poff + _L * k, _L)]
        return carry

    lax.fori_loop(0, _BPW // _L, _grp, 0)
    pltpu.sync_copy(out_v, out_hbm.at[pl.ds(base, _BPW)])


def kernel(ids, prev_inp_summ, emb_table, mapper):
    emb2 = emb_table.reshape(emb_table.shape[0] // 2, 2 * DIM)
    prev2 = prev_inp_summ.reshape(BATCH // 2, 2 * ENCDIM)
    return _dil_kernel(ids.astype(jnp.int32), prev2, emb2,
                       mapper.astype(jnp.int32))
